# Initial kernel scaffold; baseline (speedup 1.0000x reference)
import jax
import jax.numpy as jnp
from jax.experimental import pallas as pl


def kernel(x0, W0, root0, W1, root1, edge_index, edge_type):
    raise NotImplementedError("write your pallas kernel here")


# speedup vs baseline: 2.5122x; 2.5122x over previous
import functools

import jax
import jax.numpy as jnp
from jax import lax
from jax.experimental import pallas as pl
from jax.experimental.pallas import tpu as pltpu
from jax.experimental.pallas import tpu_sc as plsc

N = 10000
EMB = 128
R = 16
E = 320000
NC, NS, L = 2, 16, 16
NW = NC * NS
CHUNK = 128
CPT = 80
EP = NW * CPT * CHUNK
ZR = 632
ZR_LAST = N - ZR * (NS - 1)
BN = 1000
NB = N // BN

_MESH = plsc.VectorSubcoreMesh(core_axis_name="c", subcore_axis_name="s")


def _f32(shape):
    return jax.ShapeDtypeStruct(shape, jnp.float32)


def _piecewise(copy_one, o, n):
    for k in range(n // 128):
        copy_one(o + k * 128, 128)
    if n % 128:
        copy_one(o + (n // 128) * 128, n % 128)


def _rows_copy(copy_one, s):
    @pl.when(s < NS - 1)
    def _():
        o = pl.multiple_of(s * ZR, 8)
        _piecewise(copy_one, o, ZR)

    @pl.when(s == NS - 1)
    def _():
        _piecewise(copy_one, ZR * (NS - 1), ZR_LAST)


def _zero_vmem_2d(ref, nrows):
    zrow = jnp.zeros((16,), jnp.float32)

    def _z(i, carry):
        for k in range(EMB // 16):
            ref[i, pl.ds(k * 16, 16)] = zrow
        return carry
    lax.fori_loop(0, nrows, _z, None)


def _setup_body(src_h, dst_h, rel_h, val_h,
                g_h, w_h, wd_h,
                srcv, dstv, relv, valv, segv, cv, cdv, gv, wv, wdv, zb,
                cnt1, cntd, sem):
    c = lax.axis_index("c")
    s = lax.axis_index("s")
    wid = c * NS + s

    zrow = jnp.zeros((16,), jnp.float32)

    def _zz(i, carry):
        zb[pl.ds(i * 16, 16)] = zrow
        return carry
    lax.fori_loop(0, 2000 // 16, _zz, None)
    per = N * R // NS
    for k in range(per // 2000):
        pltpu.sync_copy(zb, cnt1.at[pl.ds(s * per + k * 2000, 2000)])
    _rows_copy(lambda o, n: pltpu.sync_copy(zb.at[pl.ds(0, n)],
                                            cntd.at[pl.ds(o, n)]), s)
    plsc.subcore_barrier()

    def _seg_of(j):
        for b in range(CHUNK // 16):
            sl = pl.ds(b * 16, 16)
            segv[sl] = dstv[sl] * R + relv[sl]

    def _hist_chunk(row, j):
        pltpu.sync_copy(dst_h.at[row, j, 0], dstv)
        pltpu.sync_copy(rel_h.at[row, j, 0], relv)
        pltpu.sync_copy(val_h.at[row, j, 0], valv)
        _seg_of(j)
        pltpu.sync_copy(valv, cnt1.at[segv], add=True)
        pltpu.sync_copy(valv, cntd.at[dstv], add=True)

    def _hist_loop(j, carry):
        _hist_chunk(s, j)
        _hist_chunk(s + NS, j)
        return carry
    lax.fori_loop(0, CPT, _hist_loop, None)
    plsc.subcore_barrier()

    def _w_loop(j, carry):
        pltpu.sync_copy(src_h.at[wid, j, 0], srcv)
        pltpu.sync_copy(dst_h.at[wid, j, 0], dstv)
        pltpu.sync_copy(rel_h.at[wid, j, 0], relv)
        pltpu.sync_copy(val_h.at[wid, j, 0], valv)
        _seg_of(j)
        pltpu.async_copy(cnt1.at[segv], cv, sem).wait()
        pltpu.async_copy(cntd.at[dstv], cdv, sem).wait()
        for b in range(CHUNK // 16):
            sl = pl.ds(b * 16, 16)
            wv[sl] = valv[sl] / jnp.maximum(cv[sl], 1.0)
            wdv[sl] = valv[sl] / jnp.maximum(cdv[sl], 1.0)
            gv[sl] = relv[sl] * N + srcv[sl]
        pltpu.sync_copy(gv, g_h.at[wid, j, 0])
        pltpu.sync_copy(wv, w_h.at[wid, j, 0])
        pltpu.sync_copy(wdv, wd_h.at[wid, j, 0])
        return carry
    lax.fori_loop(0, CPT, _w_loop, None)


_SC_PARAMS = pltpu.CompilerParams(needs_layout_passes=False)

_sc_setup = functools.partial(
    pl.kernel,
    compiler_params=_SC_PARAMS,
    out_type=[
        jax.ShapeDtypeStruct((NW, CPT, 1, CHUNK), jnp.int32),
        _f32((NW, CPT, 1, CHUNK)),
        _f32((NW, CPT, 1, CHUNK)),
    ],
    mesh=_MESH,
    scratch_types=[
        pltpu.VMEM((CHUNK,), jnp.int32),
        pltpu.VMEM((CHUNK,), jnp.int32),
        pltpu.VMEM((CHUNK,), jnp.int32),
        pltpu.VMEM((CHUNK,), jnp.float32),
        pltpu.VMEM((CHUNK,), jnp.int32),
        pltpu.VMEM((CHUNK,), jnp.float32),
        pltpu.VMEM((CHUNK,), jnp.float32),
        pltpu.VMEM((CHUNK,), jnp.int32),
        pltpu.VMEM((CHUNK,), jnp.float32),
        pltpu.VMEM((CHUNK,), jnp.float32),
        pltpu.VMEM((2000,), jnp.float32),
        pltpu.VMEM_SHARED((N * R,), jnp.float32),
        pltpu.VMEM_SHARED((N,), jnp.float32),
        pltpu.SemaphoreType.DMA,
    ],
)(_setup_body)


def _edge_body(pos, table_h, g_h, dst_h, w_h, out_h,
               gA, dA, wA, gB, dB, wB, rowsA, rowsB, acc, semA, semB):
    c = lax.axis_index("c")
    s = lax.axis_index("s")
    wid = c * NS + s

    _zero_vmem_2d(rowsA, CHUNK)
    _rows_copy(lambda o, n: pltpu.sync_copy(rowsA.at[pl.ds(0, n)],
                                            acc.at[pl.ds(o, n)]), s)
    plsc.subcore_barrier()

    def _fetch_idx(j, g, d, w):
        pltpu.sync_copy(g_h.at[wid, j, 0], g)
        pltpu.sync_copy(dst_h.at[wid, j, 0], d)
        pltpu.sync_copy(w_h.at[wid, j, 0], w)

    def _process(j, g, d, w, rows, sem, gn, dn, wn, rowsn, semn):
        @pl.when(j + 1 < CPT)
        def _():
            _fetch_idx(j + 1, gn, dn, wn)
            pltpu.async_copy(table_h.at[gn], rowsn, semn)
        pltpu.make_async_copy(table_h.at[g], rows, sem).wait()
        def _scale(i, carry):
            wv = plsc.load_gather(w, [jnp.full((16,), i, jnp.int32)])
            for k in range(EMB // 16):
                rv = rows[i, pl.ds(k * 16, 16)]
                if pos:
                    rows[i, pl.ds(k * 16, 16)] = jnp.where(rv > 0.0, wv, 0.0)
                else:
                    rows[i, pl.ds(k * 16, 16)] = rv * wv
            return carry
        lax.fori_loop(0, CHUNK, _scale, None)
        pltpu.sync_copy(rows, acc.at[d], add=True)

    _fetch_idx(0, gA, dA, wA)
    pltpu.async_copy(table_h.at[gA], rowsA, semA)

    def _pair(jj, carry):
        j = jj * 2
        _process(j, gA, dA, wA, rowsA, semA, gB, dB, wB, rowsB, semB)
        _process(j + 1, gB, dB, wB, rowsB, semB, gA, dA, wA, rowsA, semA)
        return carry
    lax.fori_loop(0, CPT // 2, _pair, None)

    plsc.subcore_barrier()

    def _wb(o, n):
        pltpu.sync_copy(acc.at[pl.ds(o, n)], rowsA.at[pl.ds(0, n)])
        pltpu.sync_copy(rowsA.at[pl.ds(0, n)], out_h.at[c, pl.ds(o, n)])
    _rows_copy(_wb, s)


def _make_edge_pass(pos):
    return functools.partial(
        pl.kernel,
        compiler_params=_SC_PARAMS,
        out_type=[_f32((NC, N, EMB))],
        mesh=_MESH,
        scratch_types=[
            pltpu.VMEM((CHUNK,), jnp.int32),
            pltpu.VMEM((CHUNK,), jnp.int32),
            pltpu.VMEM((CHUNK,), jnp.float32),
            pltpu.VMEM((CHUNK,), jnp.int32),
            pltpu.VMEM((CHUNK,), jnp.int32),
            pltpu.VMEM((CHUNK,), jnp.float32),
            pltpu.VMEM((CHUNK, EMB), jnp.float32),
            pltpu.VMEM((CHUNK, EMB), jnp.float32),
            pltpu.VMEM_SHARED((N, EMB), jnp.float32),
            pltpu.SemaphoreType.DMA,
            pltpu.SemaphoreType.DMA,
        ],
    )(functools.partial(_edge_body, pos))


_conv_pass = _make_edge_pass(False)
_ppv_pass = _make_edge_pass(True)


def _mm_body(nadd, relu, has_xout, *refs):
    xs = refs[:nadd]
    w_ref = refs[nadd]
    y_ref = refs[nadd + 1]
    x = xs[0][...]
    for a in xs[1:]:
        x = x + a[...]
    if has_xout:
        xout_ref = refs[nadd + 2]

        @pl.when(pl.program_id(1) == 0)
        def _():
            xout_ref[...] = x
    xm = jnp.maximum(x, 0.0) if relu else x
    y_ref[0] = jnp.dot(xm, w_ref[0], preferred_element_type=jnp.float32)


def _make_mm(nadd, relu, has_xout):
    in_specs = [pl.BlockSpec((BN, EMB), lambda nb, r: (nb, 0))
                for _ in range(nadd)]
    in_specs.append(pl.BlockSpec((1, EMB, EMB), lambda nb, r: (r, 0, 0)))
    out_specs = [pl.BlockSpec((1, BN, EMB), lambda nb, r: (r, nb, 0))]
    out_shape = [_f32((R + 1, N, EMB))]
    if has_xout:
        out_specs.append(pl.BlockSpec((BN, EMB), lambda nb, r: (nb, 0)))
        out_shape.append(_f32((N, EMB)))
    return pl.pallas_call(
        functools.partial(_mm_body, nadd, relu, has_xout),
        grid=(NB, R + 1),
        in_specs=in_specs,
        out_specs=out_specs if has_xout else out_specs[0],
        out_shape=out_shape if has_xout else out_shape[0],
    )


_mm0 = _make_mm(1, False, False)
_mm1x = _make_mm(3, True, True)
_mm1p = _make_mm(2, False, True)


def _add3_body(a, b, c, o):
    o[...] = a[...] + b[...] + c[...]


_add3 = pl.pallas_call(
    _add3_body,
    grid=(NB,),
    in_specs=[pl.BlockSpec((BN, EMB), lambda nb: (nb, 0))] * 3,
    out_specs=pl.BlockSpec((BN, EMB), lambda nb: (nb, 0)),
    out_shape=_f32((N, EMB)),
)


def _final_body(a, b, c, d, e, o):
    o[:, :EMB] = a[...] + b[...] + c[...]
    o[:, EMB:] = d[...] + e[...]


_final = pl.pallas_call(
    _final_body,
    grid=(NB,),
    in_specs=[pl.BlockSpec((BN, EMB), lambda nb: (nb, 0))] * 5,
    out_specs=pl.BlockSpec((BN, 2 * EMB), lambda nb: (nb, 0)),
    out_shape=_f32((N, 2 * EMB)),
)


def kernel(x0, W0, root0, W1, root1, edge_index, edge_type):
    src = edge_index[0]
    dst = edge_index[1]
    rel = edge_type
    padi = jnp.zeros((EP - E,), jnp.int32)
    srcp = jnp.concatenate([src, padi]).reshape(NW, CPT, 1, CHUNK)
    dstp = jnp.concatenate([dst, padi]).reshape(NW, CPT, 1, CHUNK)
    relp = jnp.concatenate([rel, padi]).reshape(NW, CPT, 1, CHUNK)
    val = jnp.concatenate([jnp.ones((E,), jnp.float32),
                           jnp.zeros((EP - E,), jnp.float32)]
                          ).reshape(NW, CPT, 1, CHUNK)
    g, w, wd = _sc_setup(srcp, dstp, relp, val)

    Wc0 = jnp.concatenate([W0, root0[None]], axis=0)
    Wc1 = jnp.concatenate([W1, root1[None]], axis=0)

    Y0 = _mm0(x0, Wc0)
    P0 = _conv_pass(Y0.reshape((R + 1) * N, EMB), g, dstp, w)[0]
    Y1, x1 = _mm1x(P0[0], P0[1], Y0[R], Wc1)
    Pp = _ppv_pass(x1, srcp, dstp, wd)[0]
    P1 = _conv_pass(Y1.reshape((R + 1) * N, EMB), g, dstp, w)[0]
    Yp, _ppv1 = _mm1p(Pp[0], Pp[1], Wc1)
    Pq = _conv_pass(Yp.reshape((R + 1) * N, EMB), g, dstp, w)[0]
    p2 = _add3(Pq[0], Pq[1], Yp[R])
    Pr = _ppv_pass(p2, srcp, dstp, wd)[0]

    return _final(P1[0], P1[1], Y1[R], Pr[0], Pr[1])
